# baseline (speedup 1.0000x reference)
import functools

import jax
import jax.numpy as jnp
from jax import lax
from jax.experimental import pallas as pl
from jax.experimental.pallas import tpu as pltpu
from jax.experimental.pallas import tpu_sc as plsc

B = 4096
HIST = 200
D = 64

_NC, _NS = 2, 16
_NW = _NC * _NS
_CH = 128
_NB = 4
_SETS = 2
_SG = _CH * _NB * _SETS

_NQ = B * HIST // _NW
_NT = B * D // _NW


def _gather_stream(tab, idx_v, out, row0, chunk0, nsuper, bufs, gsem, wsem):
    def body(g, carry):
        base = g * _SG
        for s in range(_SETS):
            sbase = base + s * _NB * _CH

            @pl.when(g > 0)
            def _():
                for b in range(_NB):
                    pltpu.make_async_copy(
                        bufs.at[s].at[b],
                        out.at[pl.ds(row0, _CH)],
                        wsem.at[s],
                    ).wait()

            handles = []
            for b in range(_NB):
                lc = chunk0 + g * (_SETS * _NB) + s * _NB + b
                h = pltpu.make_async_copy(
                    tab.at[idx_v.at[lc]], bufs.at[s].at[b], gsem.at[s])
                h.start()
                handles.append(h)
            for h in handles:
                h.wait()
            for b in range(_NB):
                crow = row0 + sbase + b * _CH
                pltpu.make_async_copy(
                    bufs.at[s].at[b], out.at[pl.ds(crow, _CH)], wsem.at[s]
                ).start()
        return carry

    lax.fori_loop(0, nsuper, body, 0)
    for s in range(_SETS):
        for b in range(_NB):
            pltpu.make_async_copy(
                bufs.at[s].at[b], out.at[pl.ds(row0, _CH)], wsem.at[s]
            ).wait()


def _sc_gather_body(qidx, tidx, qtab, ttab, qout, tout,
                    idx_v, bufs, gsem, wsem):
    wid = lax.axis_index("s") * _NC + lax.axis_index("c")
    nqc = _NQ // _CH
    ntc = _NT // _CH
    pltpu.sync_copy(qidx.at[pl.ds(wid * nqc, nqc)], idx_v.at[pl.ds(0, nqc)])
    pltpu.sync_copy(tidx.at[pl.ds(wid * ntc, ntc)],
                    idx_v.at[pl.ds(nqc, ntc)])
    _gather_stream(qtab, idx_v, qout, wid * _NQ, 0, _NQ // _SG,
                   bufs, gsem, wsem)
    _gather_stream(ttab, idx_v, tout, wid * _NT, nqc, _NT // _SG,
                   bufs, gsem, wsem)


def _sc_gather(qidx2d, tidx2d, qtab, ttab):
    mesh = plsc.VectorSubcoreMesh(core_axis_name="c", subcore_axis_name="s")
    return pl.kernel(
        _sc_gather_body,
        out_type=(
            jax.ShapeDtypeStruct((B * HIST, D), jnp.bfloat16),
            jax.ShapeDtypeStruct((B * D, D), jnp.bfloat16),
        ),
        mesh=mesh,
        compiler_params=pltpu.CompilerParams(use_tc_tiling_on_sc=False),
        scratch_types=[
            pltpu.VMEM(((_NQ + _NT) // _CH, _CH), jnp.int32),
            pltpu.VMEM((_SETS, _NB, _CH, D), jnp.bfloat16),
            pltpu.SemaphoreType.DMA((_SETS,)),
            pltpu.SemaphoreType.DMA((_SETS,)),
        ],
    )(qidx2d, tidx2d, qtab, ttab)


_G = 8
_QR = HIST * D // 128
_TR = D * D // 128


def _bmm_body(q_ref, t_ref, o_ref, bd_ref):
    @pl.when(pl.program_id(0) == 0)
    def _():
        bd_ref[...] = jnp.zeros((128, 128), jnp.bfloat16)

    for i in range(_G):
        t2 = t_ref[pl.ds(i * _TR, _TR), :]
        bd_ref[0:32, 0:D] = t2[:, :D]
        bd_ref[32:D, 0:D] = t2[:, D:]
        bd_ref[D:96, D:128] = t2[:, :D]
        bd_ref[96:128, D:128] = t2[:, D:]
        acc = jnp.dot(q_ref[pl.ds(i * _QR, _QR), :], bd_ref[...],
                      preferred_element_type=jnp.float32)
        o_ref[i, 0:_QR, :] = acc[:, :D]
        o_ref[i, _QR:HIST, :] = acc[:, D:]


def _tc_bmm(q2, t2):
    return pl.pallas_call(
        _bmm_body,
        grid=(B // _G,),
        in_specs=[
            pl.BlockSpec((_G * _QR, 128), lambda g: (g, 0)),
            pl.BlockSpec((_G * _TR, 128), lambda g: (g, 0)),
        ],
        out_specs=pl.BlockSpec((_G, HIST, D), lambda g: (g, 0, 0)),
        out_shape=jax.ShapeDtypeStruct((B, HIST, D), jnp.float32),
        scratch_shapes=[pltpu.VMEM((128, 128), jnp.bfloat16)],
    )(q2, t2)


def kernel(user, item_i, item_j, user_bias_table, item_bias_table,
           query_table, target_table):
    pj = item_j.reshape(B, 2, _QR).transpose(0, 2, 1)
    pt = item_i.reshape(B, 2, _TR).transpose(0, 2, 1)
    qidx = pj.reshape(-1, _CH)
    tidx = pt.reshape(-1, _CH)
    qt16 = query_table.astype(jnp.bfloat16)
    tt16 = target_table.astype(jnp.bfloat16)
    q_gath, t_gath = _sc_gather(qidx, tidx, qt16, tt16)
    return _tc_bmm(q_gath.reshape(B * _QR, 128), t_gath.reshape(B * _TR, 128))


# speedup vs baseline: 1.4249x; 1.4249x over previous
import functools

import jax
import jax.numpy as jnp
from jax import lax
from jax.experimental import pallas as pl
from jax.experimental.pallas import tpu as pltpu
from jax.experimental.pallas import tpu_sc as plsc

B = 4096
HIST = 200
D = 64

_NC, _NS = 2, 16
_NW = _NC * _NS
_CH = 128
_NB = 2
_SETS = 2
_SG = _CH * _NB * _SETS

_NQ = B * HIST // _NW
_NT = B * D // _NW


def _gather_stream(tab, idx_v, out, row0, chunk0, nsuper, bufs, gsem, wsem):
    def body(g, carry):
        base = g * _SG
        for s in range(_SETS):
            sbase = base + s * _NB * _CH

            @pl.when(g > 0)
            def _():
                for b in range(_NB):
                    pltpu.make_async_copy(
                        bufs.at[s].at[b],
                        out.at[pl.ds(row0, _CH)],
                        wsem.at[s],
                    ).wait()

            handles = []
            for b in range(_NB):
                lc = chunk0 + g * (_SETS * _NB) + s * _NB + b
                h = pltpu.make_async_copy(
                    tab.at[idx_v.at[lc]], bufs.at[s].at[b], gsem.at[s])
                h.start()
                handles.append(h)
            for h in handles:
                h.wait()
            for b in range(_NB):
                crow = row0 + sbase + b * _CH
                pltpu.make_async_copy(
                    bufs.at[s].at[b], out.at[pl.ds(crow, _CH)], wsem.at[s]
                ).start()
        return carry

    lax.fori_loop(0, nsuper, body, 0)
    for s in range(_SETS):
        for b in range(_NB):
            pltpu.make_async_copy(
                bufs.at[s].at[b], out.at[pl.ds(row0, _CH)], wsem.at[s]
            ).wait()


def _sc_gather_body(qidx, tidx, qtab, ttab, qout, tout,
                    idx_v, bufs, gsem, wsem):
    wid = lax.axis_index("s") * _NC + lax.axis_index("c")
    nqc = _NQ // _CH
    ntc = _NT // _CH
    pltpu.sync_copy(qidx.at[pl.ds(wid * nqc, nqc)], idx_v.at[pl.ds(0, nqc)])
    pltpu.sync_copy(tidx.at[pl.ds(wid * ntc, ntc)],
                    idx_v.at[pl.ds(nqc, ntc)])
    _gather_stream(qtab, idx_v, qout, wid * _NQ, 0, _NQ // _SG,
                   bufs, gsem, wsem)
    _gather_stream(ttab, idx_v, tout, wid * _NT, nqc, _NT // _SG,
                   bufs, gsem, wsem)


def _sc_gather(qidx2d, tidx2d, qtab, ttab):
    mesh = plsc.VectorSubcoreMesh(core_axis_name="c", subcore_axis_name="s")
    return pl.kernel(
        _sc_gather_body,
        out_type=(
            jax.ShapeDtypeStruct((B * HIST, 128), jnp.float32),
            jax.ShapeDtypeStruct((B * D, 128), jnp.float32),
        ),
        mesh=mesh,
        compiler_params=pltpu.CompilerParams(use_tc_tiling_on_sc=True),
        scratch_types=[
            pltpu.VMEM(((_NQ + _NT) // _CH, _CH), jnp.int32),
            pltpu.VMEM((_SETS, _NB, _CH, 128), jnp.float32),
            pltpu.SemaphoreType.DMA((_SETS,)),
            pltpu.SemaphoreType.DMA((_SETS,)),
        ],
    )(qidx2d, tidx2d, qtab, ttab)


_G = 8


def _bmm_body(q_ref, t_ref, o_ref):
    for i in range(_G):
        qv = q_ref[pl.ds(i * HIST, HIST), :D].astype(jnp.bfloat16)
        tv = t_ref[pl.ds(i * D, D), :D].astype(jnp.bfloat16)
        o_ref[i] = jnp.dot(qv, tv, preferred_element_type=jnp.float32)


def _tc_bmm(q2, t2):
    return pl.pallas_call(
        _bmm_body,
        grid=(B // _G,),
        in_specs=[
            pl.BlockSpec((_G * HIST, 128), lambda g: (g, 0)),
            pl.BlockSpec((_G * D, 128), lambda g: (g, 0)),
        ],
        out_specs=pl.BlockSpec((_G, HIST, D), lambda g: (g, 0, 0)),
        out_shape=jax.ShapeDtypeStruct((B, HIST, D), jnp.float32),
    )(q2, t2)


def kernel(user, item_i, item_j, user_bias_table, item_bias_table,
           query_table, target_table):
    qidx = item_j.reshape(-1, _CH)
    tidx = item_i.reshape(-1, _CH)
    qt = jnp.pad(query_table, ((0, 0), (0, 128 - D)))
    tt = jnp.pad(target_table, ((0, 0), (0, 128 - D)))
    q_gath, t_gath = _sc_gather(qidx, tidx, qt, tt)
    return _tc_bmm(q_gath, t_gath)


# speedup vs baseline: 1.5380x; 1.0793x over previous
import functools

import jax
import jax.numpy as jnp
from jax import lax
from jax.experimental import pallas as pl
from jax.experimental.pallas import tpu as pltpu
from jax.experimental.pallas import tpu_sc as plsc

B = 4096
HIST = 200
D = 64

_NC, _NS = 2, 16
_NW = _NC * _NS
_CH = 128
_NB = 2
_SETS = 2
_SG = _CH * _NB * _SETS

_NQ = B * HIST // _NW
_NT = B * D // _NW


def _gather_stream(tab, idx_v, out, row0, chunk0, nsuper, bufs, gsem, wsem):
    def body(g, carry):
        base = g * _SG
        for s in range(_SETS):
            sbase = base + s * _NB * _CH

            @pl.when(g > 0)
            def _():
                for b in range(_NB):
                    pltpu.make_async_copy(
                        bufs.at[s].at[b],
                        out.at[pl.ds(row0, _CH)],
                        wsem.at[s],
                    ).wait()

            handles = []
            for b in range(_NB):
                lc = chunk0 + g * (_SETS * _NB) + s * _NB + b
                h = pltpu.make_async_copy(
                    tab.at[idx_v.at[lc]], bufs.at[s].at[b], gsem.at[s])
                h.start()
                handles.append(h)
            for h in handles:
                h.wait()
            for b in range(_NB):
                crow = row0 + sbase + b * _CH
                pltpu.make_async_copy(
                    bufs.at[s].at[b], out.at[pl.ds(crow, _CH)], wsem.at[s]
                ).start()
        return carry

    lax.fori_loop(0, nsuper, body, 0)
    for s in range(_SETS):
        for b in range(_NB):
            pltpu.make_async_copy(
                bufs.at[s].at[b], out.at[pl.ds(row0, _CH)], wsem.at[s]
            ).wait()


def _sc_gather_body(nrows, idx2d, tab, out, idx_v, bufs, gsem, wsem):
    wid = lax.axis_index("s") * _NC + lax.axis_index("c")
    nc = nrows // _CH
    pltpu.sync_copy(idx2d.at[pl.ds(wid * nc, nc)], idx_v.at[pl.ds(0, nc)])
    _gather_stream(tab, idx_v, out, wid * nrows, 0, nrows // _SG,
                   bufs, gsem, wsem)


def _sc_gather(idx2d, tab, total_rows):
    nrows = total_rows // _NW
    mesh = plsc.VectorSubcoreMesh(core_axis_name="c", subcore_axis_name="s")
    return pl.kernel(
        functools.partial(_sc_gather_body, nrows),
        out_type=jax.ShapeDtypeStruct((total_rows, 128), jnp.float32),
        mesh=mesh,
        compiler_params=pltpu.CompilerParams(use_tc_tiling_on_sc=True),
        scratch_types=[
            pltpu.VMEM((nrows // _CH, _CH), jnp.int32),
            pltpu.VMEM((_SETS, _NB, _CH, 128), jnp.float32),
            pltpu.SemaphoreType.DMA((_SETS,)),
            pltpu.SemaphoreType.DMA((_SETS,)),
        ],
    )(idx2d, tab)


_G = 16


def _bmm_body(q_ref, t_ref, o_ref):
    for i in range(_G):
        qv = q_ref[pl.ds(i * HIST, HIST), :D].astype(jnp.bfloat16)
        tv = t_ref[pl.ds(i * D, D), :D].astype(jnp.bfloat16)
        o_ref[i] = jnp.dot(qv, tv, preferred_element_type=jnp.float32)


def _tc_bmm(q2, t2):
    return pl.pallas_call(
        _bmm_body,
        grid=(B // _G,),
        in_specs=[
            pl.BlockSpec((_G * HIST, 128), lambda g: (g, 0)),
            pl.BlockSpec((_G * D, 128), lambda g: (g, 0)),
        ],
        out_specs=pl.BlockSpec((_G, HIST, D), lambda g: (g, 0, 0)),
        out_shape=jax.ShapeDtypeStruct((B, HIST, D), jnp.float32),
    )(q2, t2)


def kernel(user, item_i, item_j, user_bias_table, item_bias_table,
           query_table, target_table):
    qidx = item_j.reshape(-1, _CH)
    tidx = item_i.reshape(-1, _CH)
    qt = jnp.pad(query_table, ((0, 0), (0, 128 - D)))
    tt = jnp.pad(target_table, ((0, 0), (0, 128 - D)))
    q_gath = _sc_gather(qidx, qt, B * HIST)
    t_gath = _sc_gather(tidx, tt, B * D)
    return _tc_bmm(q_gath, t_gath)
